# Initial kernel scaffold; baseline (speedup 1.0000x reference)
import jax
import jax.numpy as jnp
from jax.experimental import pallas as pl


def kernel(x, edge_attr, descriptors, Wn, bnb, gn, btn, We, be, ge, bte, Wc0, bc0, gc0, btc0, Wc1, bc1, gc1, btc1, Wc2, bc2, gc2, btc2, Wd, bd, gd, btd, Wf1, bf1, Wf2, bf2, edge_index, batch):
    raise NotImplementedError("write your pallas kernel here")


# speedup vs baseline: 4.2989x; 4.2989x over previous
import functools

import jax
import jax.numpy as jnp
from jax import lax
from jax.experimental import pallas as pl
from jax.experimental.pallas import tpu as pltpu
from jax.experimental.pallas import tpu_sc as plsc

_N = 10000
_E = 320000
_G = 64
_DIN = 128
_DE = 16
_H = 64
_EPS = 1e-5

_NC = 2
_NS = 16
_NW = _NC * _NS
_C = 80
_EPW = _E // _NW
_CH = _EPW // _C
_NP = 10240
_RPS = _NP // _NS
_ZR = 128


def _full(shape):
    return pl.BlockSpec(shape, lambda *args: (0,) * len(shape))


def _edge_stats_body(ea_ref, We_ref, be_ref, out_ref):
    i = pl.program_id(0)
    y = jnp.maximum(
        jnp.dot(ea_ref[...], We_ref[...], preferred_element_type=jnp.float32)
        + be_ref[...], 0.0)
    part = jnp.concatenate(
        [jnp.sum(y, axis=0).reshape(1, _H),
         jnp.sum(y * y, axis=0).reshape(1, _H)], axis=0)

    @pl.when(i == 0)
    def _():
        out_ref[...] = part

    @pl.when(i > 0)
    def _():
        out_ref[...] += part


def _edge_stats(edge_attr, We, be):
    nb = 40
    rows = _E // nb
    return pl.pallas_call(
        _edge_stats_body,
        grid=(nb,),
        in_specs=[
            pl.BlockSpec((rows, _DE), lambda i: (i, 0)),
            _full((_DE, _H)),
            _full((_H,)),
        ],
        out_specs=_full((2, _H)),
        out_shape=jax.ShapeDtypeStruct((2, _H), jnp.float32),
    )(edge_attr, We, be)


def _edge_emit_body(ea_ref, st_ref, We_ref, be_ref, ge_ref, bte_ref, out_ref):
    mean = st_ref[0, :] * (1.0 / _E)
    var = st_ref[1, :] * (1.0 / _E) - mean * mean
    scale = ge_ref[...] * lax.rsqrt(var + _EPS)
    shift = bte_ref[...] - mean * scale
    y = jnp.maximum(
        jnp.dot(ea_ref[...], We_ref[...], preferred_element_type=jnp.float32)
        + be_ref[...], 0.0)
    out_ref[...] = jnp.maximum(y * scale + shift, 0.0)


def _edge_emit(edge_attr, stats, We, be, ge, bte):
    nb = 40
    rows = _E // nb
    return pl.pallas_call(
        _edge_emit_body,
        grid=(nb,),
        in_specs=[
            pl.BlockSpec((rows, _DE), lambda i: (i, 0)),
            _full((2, _H)),
            _full((_DE, _H)),
            _full((_H,)),
            _full((_H,)),
            _full((_H,)),
        ],
        out_specs=pl.BlockSpec((rows, _H), lambda i: (i, 0)),
        out_shape=jax.ShapeDtypeStruct((_E, _H), jnp.float32),
    )(edge_attr, stats, We, be, ge, bte)


def _enc_rows(z, W, b, g, bt):
    y = jnp.maximum(
        jnp.dot(z, W, preferred_element_type=jnp.float32) + b, 0.0)
    m = jnp.mean(y, axis=0)
    d = y - m
    v = jnp.mean(d * d, axis=0)
    return jnp.maximum(d * lax.rsqrt(v + _EPS) * g + bt, 0.0)


def _node_enc_body(x_ref, W_ref, b_ref, g_ref, bt_ref, out_ref):
    out_ref[...] = _enc_rows(x_ref[...], W_ref[...], b_ref[...], g_ref[...],
                             bt_ref[...])


def _node_enc(x, Wn, bnb, gn, btn):
    return pl.pallas_call(
        _node_enc_body,
        in_specs=[_full((_N, _DIN)), _full((_DIN, _H)), _full((_H,)),
                  _full((_H,)), _full((_H,))],
        out_specs=_full((_N, _H)),
        out_shape=jax.ShapeDtypeStruct((_N, _H), jnp.float32),
    )(x, Wn, bnb, gn, btn)


def _layer_enc_body(h_ref, parts_ref, W_ref, b_ref, g_ref, bt_ref, out_ref):
    z = h_ref[...] + parts_ref[0, :_N, :] + parts_ref[1, :_N, :]
    out_ref[...] = _enc_rows(z, W_ref[...], b_ref[...], g_ref[...],
                             bt_ref[...])


def _layer_enc(h, parts, W, b, g, bt):
    return pl.pallas_call(
        _layer_enc_body,
        in_specs=[_full((_N, _H)), _full((2, _NP, _H)), _full((_H, _H)),
                  _full((_H,)), _full((_H,)), _full((_H,))],
        out_specs=_full((_N, _H)),
        out_shape=jax.ShapeDtypeStruct((_N, _H), jnp.float32),
    )(h, parts, W, b, g, bt)


def _final_body(h_ref, batch_ref, desc_ref, Wd_ref, bd_ref, gd_ref, btd_ref,
                Wf1_ref, bf1_ref, Wf2_ref, bf2_ref, out_ref):
    seg = lax.broadcasted_iota(jnp.int32, (_G, _N), 0)
    onehot = jnp.where(seg == batch_ref[0, :][None, :], 1.0, 0.0)
    pooled = jnp.dot(onehot, h_ref[...], preferred_element_type=jnp.float32)
    demb = _enc_rows(desc_ref[...], Wd_ref[...], bd_ref[...], gd_ref[...],
                     btd_ref[...])
    comb = jnp.concatenate([pooled, demb], axis=1)
    o = jnp.maximum(
        jnp.dot(comb, Wf1_ref[...], preferred_element_type=jnp.float32)
        + bf1_ref[...], 0.0)
    o2 = jnp.dot(o, Wf2_ref[...], preferred_element_type=jnp.float32) \
        + bf2_ref[...]
    out_ref[...] = 1.0 / (1.0 + jnp.exp(-o2))


def _final(h, batch2d, desc, Wd, bd, gd, btd, Wf1, bf1, Wf2, bf2):
    return pl.pallas_call(
        _final_body,
        in_specs=[_full((_N, _H)), _full((1, _N)), _full((_G, _DE)),
                  _full((_DE, _H)), _full((_H,)), _full((_H,)), _full((_H,)),
                  _full((2 * _H, _H)), _full((_H,)), _full((_H, 1)),
                  _full((1,))],
        out_specs=_full((_G, 1)),
        out_shape=jax.ShapeDtypeStruct((_G, 1), jnp.float32),
    )(h, batch2d, desc, Wd, bd, gd, btd, Wf1, bf1, Wf2, bf2)


def _sc_msg_body(h_hbm, e_hbm, src_hbm, dst_hbm, out_hbm,
                 src_v, dst_v, rows_v, e_v, zbuf, aggr_sh, sem1, sem2):
    c = lax.axis_index("c")
    s = lax.axis_index("s")
    wid = c * _NS + s

    pltpu.sync_copy(src_hbm.at[wid], src_v)
    pltpu.sync_copy(dst_hbm.at[wid], dst_v)

    def zrow(i, carry):
        for j in range(_H // 16):
            zbuf[i, pl.ds(j * 16, 16)] = jnp.zeros((16,), jnp.float32)
        return carry
    lax.fori_loop(0, _ZR, zrow, 0)
    for k in range(_RPS // _ZR):
        pltpu.sync_copy(zbuf, aggr_sh.at[pl.ds(s * _RPS + k * _ZR, _ZR)])
    plsc.subcore_barrier()

    def chunk(g, carry):
        off = wid * _EPW + g * _C
        gat = pltpu.async_copy(h_hbm.at[src_v.at[g]], rows_v, sem1)
        ecp = pltpu.async_copy(e_hbm.at[pl.ds(off, _C)], e_v, sem2)
        gat.wait()
        ecp.wait()

        def row(i, rcarry):
            for j in range(_H // 16):
                sl = pl.ds(j * 16, 16)
                rows_v[i, sl] = jnp.maximum(rows_v[i, sl] + e_v[i, sl], 0.0)
            return rcarry
        lax.fori_loop(0, _C, row, 0)
        pltpu.sync_copy(rows_v, aggr_sh.at[dst_v.at[g]], add=True)
        return carry
    lax.fori_loop(0, _CH, chunk, 0)

    plsc.subcore_barrier()
    pltpu.sync_copy(aggr_sh.at[pl.ds(s * _RPS, _RPS)],
                    out_hbm.at[c, pl.ds(s * _RPS, _RPS)])


@functools.lru_cache(maxsize=1)
def _sc_msg_kernel_fn():
    mesh = plsc.VectorSubcoreMesh(core_axis_name="c", subcore_axis_name="s",
                                  num_cores=_NC, num_subcores=_NS)
    return pl.kernel(
        _sc_msg_body,
        out_type=jax.ShapeDtypeStruct((2, _NP, _H), jnp.float32),
        mesh=mesh,
        scratch_types=[
            pltpu.VMEM((_CH, _C), jnp.int32),
            pltpu.VMEM((_CH, _C), jnp.int32),
            pltpu.VMEM((_C, _H), jnp.float32),
            pltpu.VMEM((_C, _H), jnp.float32),
            pltpu.VMEM((_ZR, _H), jnp.float32),
            pltpu.VMEM_SHARED((_NP, _H), jnp.float32),
            pltpu.SemaphoreType.DMA,
            pltpu.SemaphoreType.DMA,
        ],
        compiler_params=pltpu.CompilerParams(use_tc_tiling_on_sc=False),
    )


def kernel(x, edge_attr, descriptors, Wn, bnb, gn, btn, We, be, ge, bte,
           Wc0, bc0, gc0, btc0, Wc1, bc1, gc1, btc1, Wc2, bc2, gc2, btc2,
           Wd, bd, gd, btd, Wf1, bf1, Wf2, bf2, edge_index, batch):
    src3d = edge_index[0].reshape(_NW, _CH, _C)
    dst3d = edge_index[1].reshape(_NW, _CH, _C)
    batch2d = batch.reshape(1, _N)
    desc = descriptors[:, 0, :]

    stats = _edge_stats(edge_attr, We, be)
    e = _edge_emit(edge_attr, stats, We, be, ge, bte)
    h = _node_enc(x, Wn, bnb, gn, btn)

    for (W, b, g, bt) in ((Wc0, bc0, gc0, btc0), (Wc1, bc1, gc1, btc1),
                          (Wc2, bc2, gc2, btc2)):
        parts = _sc_msg_kernel_fn()(h, e, src3d, dst3d)
        h = _layer_enc(h, parts, W, b, g, bt)

    out = _final(h, batch2d, desc, Wd, bd, gd, btd, Wf1, bf1, Wf2, bf2)
    return out.reshape(-1)


# speedup vs baseline: 5.7289x; 1.3326x over previous
import functools

import jax
import jax.numpy as jnp
from jax import lax
from jax.experimental import pallas as pl
from jax.experimental.pallas import tpu as pltpu
from jax.experimental.pallas import tpu_sc as plsc

_N = 10000
_E = 320000
_G = 64
_DIN = 128
_DE = 16
_H = 64
_EPS = 1e-5

_NC = 2
_NS = 16
_NW = _NC * _NS
_C = 80
_EPW = _E // _NW
_CH = _EPW // _C
_NP = 10240
_RPS = _NP // _NS
_ZR = 128


def _full(shape):
    return pl.BlockSpec(shape, lambda *args: (0,) * len(shape))


def _edge_stats_body(ea_ref, We_ref, be_ref, out_ref):
    i = pl.program_id(0)
    y = jnp.maximum(
        jnp.dot(ea_ref[...], We_ref[...], preferred_element_type=jnp.float32)
        + be_ref[...], 0.0)
    part = jnp.concatenate(
        [jnp.sum(y, axis=0).reshape(1, _H),
         jnp.sum(y * y, axis=0).reshape(1, _H)], axis=0)

    @pl.when(i == 0)
    def _():
        out_ref[...] = part

    @pl.when(i > 0)
    def _():
        out_ref[...] += part


def _edge_stats(edge_attr, We, be):
    nb = 40
    rows = _E // nb
    return pl.pallas_call(
        _edge_stats_body,
        grid=(nb,),
        in_specs=[
            pl.BlockSpec((rows, _DE), lambda i: (i, 0)),
            _full((_DE, _H)),
            _full((_H,)),
        ],
        out_specs=_full((2, _H)),
        out_shape=jax.ShapeDtypeStruct((2, _H), jnp.float32),
    )(edge_attr, We, be)


def _edge_emit_body(ea_ref, st_ref, We_ref, be_ref, ge_ref, bte_ref, out_ref):
    mean = st_ref[0, :] * (1.0 / _E)
    var = st_ref[1, :] * (1.0 / _E) - mean * mean
    scale = ge_ref[...] * lax.rsqrt(var + _EPS)
    shift = bte_ref[...] - mean * scale
    y = jnp.maximum(
        jnp.dot(ea_ref[...], We_ref[...], preferred_element_type=jnp.float32)
        + be_ref[...], 0.0)
    out_ref[...] = jnp.maximum(y * scale + shift, 0.0)


def _edge_emit(edge_attr, stats, We, be, ge, bte):
    nb = 40
    rows = _E // nb
    return pl.pallas_call(
        _edge_emit_body,
        grid=(nb,),
        in_specs=[
            pl.BlockSpec((rows, _DE), lambda i: (i, 0)),
            _full((2, _H)),
            _full((_DE, _H)),
            _full((_H,)),
            _full((_H,)),
            _full((_H,)),
        ],
        out_specs=pl.BlockSpec((rows, _H), lambda i: (i, 0)),
        out_shape=jax.ShapeDtypeStruct((_E, _H), jnp.float32),
    )(edge_attr, stats, We, be, ge, bte)


def _enc_rows(z, W, b, g, bt):
    y = jnp.maximum(
        jnp.dot(z, W, preferred_element_type=jnp.float32) + b, 0.0)
    m = jnp.mean(y, axis=0)
    d = y - m
    v = jnp.mean(d * d, axis=0)
    return jnp.maximum(d * lax.rsqrt(v + _EPS) * g + bt, 0.0)


def _node_enc_body(x_ref, W_ref, b_ref, g_ref, bt_ref, out_ref):
    out_ref[...] = _enc_rows(x_ref[...], W_ref[...], b_ref[...], g_ref[...],
                             bt_ref[...])


def _node_enc(x, Wn, bnb, gn, btn):
    return pl.pallas_call(
        _node_enc_body,
        in_specs=[_full((_N, _DIN)), _full((_DIN, _H)), _full((_H,)),
                  _full((_H,)), _full((_H,))],
        out_specs=_full((_N, _H)),
        out_shape=jax.ShapeDtypeStruct((_N, _H), jnp.float32),
    )(x, Wn, bnb, gn, btn)


def _layer_enc_body(h_ref, parts_ref, W_ref, b_ref, g_ref, bt_ref, out_ref):
    z = h_ref[...] + parts_ref[0, :_N, :] + parts_ref[1, :_N, :]
    out_ref[...] = _enc_rows(z, W_ref[...], b_ref[...], g_ref[...],
                             bt_ref[...])


def _layer_enc(h, parts, W, b, g, bt):
    return pl.pallas_call(
        _layer_enc_body,
        in_specs=[_full((_N, _H)), _full((2, _NP, _H)), _full((_H, _H)),
                  _full((_H,)), _full((_H,)), _full((_H,))],
        out_specs=_full((_N, _H)),
        out_shape=jax.ShapeDtypeStruct((_N, _H), jnp.float32),
    )(h, parts, W, b, g, bt)


def _final_body(h_ref, batch_ref, desc_ref, Wd_ref, bd_ref, gd_ref, btd_ref,
                Wf1_ref, bf1_ref, Wf2_ref, bf2_ref, out_ref):
    seg = lax.broadcasted_iota(jnp.int32, (_G, _N), 0)
    onehot = jnp.where(seg == batch_ref[0, :][None, :], 1.0, 0.0)
    pooled = jnp.dot(onehot, h_ref[...], preferred_element_type=jnp.float32)
    demb = _enc_rows(desc_ref[...], Wd_ref[...], bd_ref[...], gd_ref[...],
                     btd_ref[...])
    comb = jnp.concatenate([pooled, demb], axis=1)
    o = jnp.maximum(
        jnp.dot(comb, Wf1_ref[...], preferred_element_type=jnp.float32)
        + bf1_ref[...], 0.0)
    o2 = jnp.dot(o, Wf2_ref[...], preferred_element_type=jnp.float32) \
        + bf2_ref[...]
    out_ref[...] = 1.0 / (1.0 + jnp.exp(-o2))


def _final(h, batch2d, desc, Wd, bd, gd, btd, Wf1, bf1, Wf2, bf2):
    return pl.pallas_call(
        _final_body,
        in_specs=[_full((_N, _H)), _full((1, _N)), _full((_G, _DE)),
                  _full((_DE, _H)), _full((_H,)), _full((_H,)), _full((_H,)),
                  _full((2 * _H, _H)), _full((_H,)), _full((_H, 1)),
                  _full((1,))],
        out_specs=_full((_G, 1)),
        out_shape=jax.ShapeDtypeStruct((_G, 1), jnp.float32),
    )(h, batch2d, desc, Wd, bd, gd, btd, Wf1, bf1, Wf2, bf2)


def _sc_msg_body(h_hbm, e_hbm, src_hbm, dst_hbm, out_hbm,
                 src_v, dst_v, rows_a, e_a, rows_b, e_b, zbuf, aggr_sh,
                 sem_ga, sem_ea, sem_gb, sem_eb):
    c = lax.axis_index("c")
    s = lax.axis_index("s")
    wid = c * _NS + s

    pltpu.sync_copy(src_hbm.at[wid], src_v)
    pltpu.sync_copy(dst_hbm.at[wid], dst_v)

    def zrow(i, carry):
        for j in range(_H // 16):
            zbuf[i, pl.ds(j * 16, 16)] = jnp.zeros((16,), jnp.float32)
        return carry
    lax.fori_loop(0, _ZR, zrow, 0)
    for k in range(_RPS // _ZR):
        pltpu.sync_copy(zbuf, aggr_sh.at[pl.ds(s * _RPS + k * _ZR, _ZR)])
    plsc.subcore_barrier()

    ebase = wid * _EPW

    def issue(g, rows_v, e_v, sem_g, sem_e):
        pltpu.async_copy(h_hbm.at[src_v.at[g]], rows_v, sem_g)
        pltpu.async_copy(e_hbm.at[pl.ds(ebase + g * _C, _C)], e_v, sem_e)

    def drain(g, rows_v, e_v, sem_g, sem_e):
        pltpu.make_async_copy(h_hbm.at[src_v.at[g]], rows_v, sem_g).wait()
        pltpu.make_async_copy(e_hbm.at[pl.ds(0, _C)], e_v, sem_e).wait()

    def process(g, rows_v, e_v, sem_g, sem_e):
        drain(g, rows_v, e_v, sem_g, sem_e)

        def _row(i, rcarry):
            for j in range(_H // 16):
                sl = pl.ds(j * 16, 16)
                rows_v[i, sl] = jnp.maximum(rows_v[i, sl] + e_v[i, sl], 0.0)
            return rcarry
        lax.fori_loop(0, _C, _row, 0)
        pltpu.sync_copy(rows_v, aggr_sh.at[dst_v.at[g]], add=True)

    issue(0, rows_a, e_a, sem_ga, sem_ea)

    def pair(i, carry):
        ga = 2 * i
        issue(ga + 1, rows_b, e_b, sem_gb, sem_eb)
        process(ga, rows_a, e_a, sem_ga, sem_ea)
        issue(ga + 2, rows_a, e_a, sem_ga, sem_ea)
        process(ga + 1, rows_b, e_b, sem_gb, sem_eb)
        return carry
    lax.fori_loop(0, (_CH - 1) // 2, pair, 0)
    process(_CH - 1, rows_a, e_a, sem_ga, sem_ea)

    plsc.subcore_barrier()
    pltpu.sync_copy(aggr_sh.at[pl.ds(s * _RPS, _RPS)],
                    out_hbm.at[c, pl.ds(s * _RPS, _RPS)])


@functools.lru_cache(maxsize=1)
def _sc_msg_kernel_fn():
    mesh = plsc.VectorSubcoreMesh(core_axis_name="c", subcore_axis_name="s",
                                  num_cores=_NC, num_subcores=_NS)
    return pl.kernel(
        _sc_msg_body,
        out_type=jax.ShapeDtypeStruct((2, _NP, _H), jnp.float32),
        mesh=mesh,
        scratch_types=[
            pltpu.VMEM((_CH, _C), jnp.int32),
            pltpu.VMEM((_CH, _C), jnp.int32),
            pltpu.VMEM((_C, _H), jnp.float32),
            pltpu.VMEM((_C, _H), jnp.float32),
            pltpu.VMEM((_C, _H), jnp.float32),
            pltpu.VMEM((_C, _H), jnp.float32),
            pltpu.VMEM((_ZR, _H), jnp.float32),
            pltpu.VMEM_SHARED((_NP, _H), jnp.float32),
            pltpu.SemaphoreType.DMA,
            pltpu.SemaphoreType.DMA,
            pltpu.SemaphoreType.DMA,
            pltpu.SemaphoreType.DMA,
        ],
        compiler_params=pltpu.CompilerParams(use_tc_tiling_on_sc=False),
    )


def kernel(x, edge_attr, descriptors, Wn, bnb, gn, btn, We, be, ge, bte,
           Wc0, bc0, gc0, btc0, Wc1, bc1, gc1, btc1, Wc2, bc2, gc2, btc2,
           Wd, bd, gd, btd, Wf1, bf1, Wf2, bf2, edge_index, batch):
    src3d = edge_index[0].reshape(_NW, _CH, _C)
    dst3d = edge_index[1].reshape(_NW, _CH, _C)
    batch2d = batch.reshape(1, _N)
    desc = descriptors[:, 0, :]

    stats = _edge_stats(edge_attr, We, be)
    e = _edge_emit(edge_attr, stats, We, be, ge, bte)
    h = _node_enc(x, Wn, bnb, gn, btn)

    for (W, b, g, bt) in ((Wc0, bc0, gc0, btc0), (Wc1, bc1, gc1, btc1),
                          (Wc2, bc2, gc2, btc2)):
        parts = _sc_msg_kernel_fn()(h, e, src3d, dst3d)
        h = _layer_enc(h, parts, W, b, g, bt)

    out = _final(h, batch2d, desc, Wd, bd, gd, btd, Wf1, bf1, Wf2, bf2)
    return out.reshape(-1)


# speedup vs baseline: 8.9343x; 1.5595x over previous
import functools

import jax
import jax.numpy as jnp
from jax import lax
from jax.experimental import pallas as pl
from jax.experimental.pallas import tpu as pltpu
from jax.experimental.pallas import tpu_sc as plsc

_N = 10000
_E = 320000
_G = 64
_DIN = 128
_DE = 16
_H = 64
_EPS = 1e-5

_NC = 2
_NS = 16
_NW = _NC * _NS
_C = 80
_EPW = _E // _NW
_CH = _EPW // _C
_NP = 10240
_RPS = _NP // _NS
_ZR = 128


def _full(shape):
    return pl.BlockSpec(shape, lambda *args: (0,) * len(shape))


def _dott(lhs_t, rhs):
    return lax.dot_general(lhs_t, rhs, (((0,), (0,)), ((), ())),
                           preferred_element_type=jnp.float32)


def _edge_stats_body(ea_ref, We_ref, be_ref, out_ref):
    i = pl.program_id(0)
    y = jnp.maximum(_dott(ea_ref[...], We_ref[...]) + be_ref[...], 0.0)
    part = jnp.concatenate(
        [jnp.sum(y, axis=0).reshape(1, _H),
         jnp.sum(y * y, axis=0).reshape(1, _H)], axis=0)

    @pl.when(i == 0)
    def _():
        out_ref[...] = part

    @pl.when(i > 0)
    def _():
        out_ref[...] += part


def _edge_stats(ea_t, We, be):
    nb = 50
    cols = _E // nb
    return pl.pallas_call(
        _edge_stats_body,
        grid=(nb,),
        in_specs=[
            pl.BlockSpec((_DE, cols), lambda i: (0, i)),
            _full((_DE, _H)),
            _full((_H,)),
        ],
        out_specs=_full((2, _H)),
        out_shape=jax.ShapeDtypeStruct((2, _H), jnp.float32),
    )(ea_t, We, be)


def _edge_emit_body(ea0_ref, ea1_ref, st_ref, We_ref, be_ref, ge_ref,
                    bte_ref, out_ref):
    mean = st_ref[0, :] * (1.0 / _E)
    var = st_ref[1, :] * (1.0 / _E) - mean * mean
    scale = ge_ref[...] * lax.rsqrt(var + _EPS)
    shift = bte_ref[...] - mean * scale
    y0 = jnp.maximum(_dott(ea0_ref[...], We_ref[...]) + be_ref[...], 0.0)
    y1 = jnp.maximum(_dott(ea1_ref[...], We_ref[...]) + be_ref[...], 0.0)
    e0 = jnp.maximum(y0 * scale + shift, 0.0)
    e1 = jnp.maximum(y1 * scale + shift, 0.0)
    out_ref[...] = jnp.concatenate([e0, e1], axis=1)


def _edge_emit(ea_t, stats, We, be, ge, bte):
    nb = 25
    cols = (_E // 2) // nb
    return pl.pallas_call(
        _edge_emit_body,
        grid=(nb,),
        in_specs=[
            pl.BlockSpec((_DE, cols), lambda i: (0, i)),
            pl.BlockSpec((_DE, cols), lambda i: (0, i + nb)),
            _full((2, _H)),
            _full((_DE, _H)),
            _full((_H,)),
            _full((_H,)),
            _full((_H,)),
        ],
        out_specs=pl.BlockSpec((cols, 2 * _H), lambda i: (i, 0)),
        out_shape=jax.ShapeDtypeStruct((_E // 2, 2 * _H), jnp.float32),
    )(ea_t, ea_t, stats, We, be, ge, bte)


def _enc_rows(z, W, b, g, bt):
    y = jnp.maximum(
        jnp.dot(z, W, preferred_element_type=jnp.float32) + b, 0.0)
    m = jnp.mean(y, axis=0)
    d = y - m
    v = jnp.mean(d * d, axis=0)
    return jnp.maximum(d * lax.rsqrt(v + _EPS) * g + bt, 0.0)


def _node_enc_body(x_ref, W_ref, b_ref, g_ref, bt_ref, out_ref):
    out_ref[...] = _enc_rows(x_ref[...], W_ref[...], b_ref[...], g_ref[...],
                             bt_ref[...])


def _node_enc(x, Wn, bnb, gn, btn):
    return pl.pallas_call(
        _node_enc_body,
        in_specs=[_full((_N, _DIN)), _full((_DIN, _H)), _full((_H,)),
                  _full((_H,)), _full((_H,))],
        out_specs=_full((_N, _H)),
        out_shape=jax.ShapeDtypeStruct((_N, _H), jnp.float32),
    )(x, Wn, bnb, gn, btn)


def _layer_enc_body(h_ref, parts_ref, W_ref, b_ref, g_ref, bt_ref, out_ref):
    z = (h_ref[...] + parts_ref[:_N // 2, :]
         + parts_ref[_NP // 2:_NP // 2 + _N // 2, :])
    zero = jnp.zeros((_H, _H), jnp.float32)
    W2 = jnp.concatenate(
        [jnp.concatenate([W_ref[...], zero], axis=1),
         jnp.concatenate([zero, W_ref[...]], axis=1)], axis=0)
    b2 = jnp.concatenate([b_ref[...], b_ref[...]])
    y = jnp.maximum(
        jnp.dot(z, W2, preferred_element_type=jnp.float32) + b2, 0.0)
    s = jnp.sum(y, axis=0)
    m64 = (s[:_H] + s[_H:]) * (1.0 / _N)
    d = y - jnp.concatenate([m64, m64])
    sq = jnp.sum(d * d, axis=0)
    v64 = (sq[:_H] + sq[_H:]) * (1.0 / _N)
    r = jnp.concatenate([lax.rsqrt(v64 + _EPS)] * 2)
    g2 = jnp.concatenate([g_ref[...], g_ref[...]])
    bt2 = jnp.concatenate([bt_ref[...], bt_ref[...]])
    out_ref[...] = jnp.maximum(d * r * g2 + bt2, 0.0)


def _layer_enc(h128, parts128, W, b, g, bt):
    return pl.pallas_call(
        _layer_enc_body,
        in_specs=[_full((_N // 2, 2 * _H)), _full((_NP, 2 * _H)),
                  _full((_H, _H)), _full((_H,)), _full((_H,)), _full((_H,))],
        out_specs=_full((_N // 2, 2 * _H)),
        out_shape=jax.ShapeDtypeStruct((_N // 2, 2 * _H), jnp.float32),
    )(h128, parts128, W, b, g, bt)


def _final_body(h_ref, batch_ref, desc_ref, Wd_ref, bd_ref, gd_ref, btd_ref,
                Wf1_ref, bf1_ref, Wf2_ref, bf2_ref, out_ref):
    seg = lax.broadcasted_iota(jnp.int32, (_G, _N), 0)
    onehot = jnp.where(seg == batch_ref[0, :][None, :], 1.0, 0.0)
    pooled = jnp.dot(onehot, h_ref[...], preferred_element_type=jnp.float32)
    demb = _enc_rows(desc_ref[...], Wd_ref[...], bd_ref[...], gd_ref[...],
                     btd_ref[...])
    comb = jnp.concatenate([pooled, demb], axis=1)
    o = jnp.maximum(
        jnp.dot(comb, Wf1_ref[...], preferred_element_type=jnp.float32)
        + bf1_ref[...], 0.0)
    o2 = jnp.dot(o, Wf2_ref[...], preferred_element_type=jnp.float32) \
        + bf2_ref[...]
    out_ref[...] = 1.0 / (1.0 + jnp.exp(-o2))


def _final(h, batch2d, desc, Wd, bd, gd, btd, Wf1, bf1, Wf2, bf2):
    return pl.pallas_call(
        _final_body,
        in_specs=[_full((_N, _H)), _full((1, _N)), _full((_G, _DE)),
                  _full((_DE, _H)), _full((_H,)), _full((_H,)), _full((_H,)),
                  _full((2 * _H, _H)), _full((_H,)), _full((_H, 1)),
                  _full((1,))],
        out_specs=_full((_G, 1)),
        out_shape=jax.ShapeDtypeStruct((_G, 1), jnp.float32),
    )(h, batch2d, desc, Wd, bd, gd, btd, Wf1, bf1, Wf2, bf2)


def _sc_msg_body(h_hbm, e_hbm, src_hbm, dst_hbm, out_hbm,
                 src_v, dst_v, rows_a, e_a, rows_b, e_b, zbuf, aggr_sh,
                 sem_ga, sem_ea, sem_gb, sem_eb):
    c = lax.axis_index("c")
    s = lax.axis_index("s")
    wid = c * _NS + s

    pltpu.sync_copy(src_hbm.at[wid], src_v)
    pltpu.sync_copy(dst_hbm.at[wid], dst_v)

    def zrow(i, carry):
        for j in range(_H // 16):
            zbuf[i, pl.ds(j * 16, 16)] = jnp.zeros((16,), jnp.float32)
        return carry
    lax.fori_loop(0, _ZR, zrow, 0)
    for k in range(_RPS // _ZR):
        pltpu.sync_copy(zbuf, aggr_sh.at[pl.ds(s * _RPS + k * _ZR, _ZR)])
    plsc.subcore_barrier()

    ebase = s * _EPW
    ecol = c * _H

    def issue(g, rows_v, e_v, sem_g, sem_e):
        pltpu.async_copy(h_hbm.at[src_v.at[g]], rows_v, sem_g)
        pltpu.async_copy(
            e_hbm.at[pl.ds(ebase + g * _C, _C), pl.ds(ecol, _H)], e_v, sem_e)

    def drain(g, rows_v, e_v, sem_g, sem_e):
        pltpu.make_async_copy(h_hbm.at[src_v.at[g]], rows_v, sem_g).wait()
        pltpu.make_async_copy(
            e_hbm.at[pl.ds(0, _C), pl.ds(ecol, _H)], e_v, sem_e).wait()

    def process(g, rows_v, e_v, sem_g, sem_e):
        drain(g, rows_v, e_v, sem_g, sem_e)

        def _row(i, rcarry):
            for j in range(_H // 16):
                sl = pl.ds(j * 16, 16)
                rows_v[i, sl] = jnp.maximum(rows_v[i, sl] + e_v[i, sl], 0.0)
            return rcarry
        lax.fori_loop(0, _C, _row, 0)
        pltpu.sync_copy(rows_v, aggr_sh.at[dst_v.at[g]], add=True)

    issue(0, rows_a, e_a, sem_ga, sem_ea)

    def pair(i, carry):
        ga = 2 * i
        issue(ga + 1, rows_b, e_b, sem_gb, sem_eb)
        process(ga, rows_a, e_a, sem_ga, sem_ea)
        issue(ga + 2, rows_a, e_a, sem_ga, sem_ea)
        process(ga + 1, rows_b, e_b, sem_gb, sem_eb)
        return carry
    lax.fori_loop(0, (_CH - 1) // 2, pair, 0)
    process(_CH - 1, rows_a, e_a, sem_ga, sem_ea)

    plsc.subcore_barrier()
    pltpu.sync_copy(aggr_sh.at[pl.ds(s * _RPS, _RPS)],
                    out_hbm.at[c, pl.ds(s * _RPS, _RPS)])


@functools.lru_cache(maxsize=1)
def _sc_msg_kernel_fn():
    mesh = plsc.VectorSubcoreMesh(core_axis_name="c", subcore_axis_name="s",
                                  num_cores=_NC, num_subcores=_NS)
    return pl.kernel(
        _sc_msg_body,
        out_type=jax.ShapeDtypeStruct((2, _NP, _H), jnp.float32),
        mesh=mesh,
        scratch_types=[
            pltpu.VMEM((_CH, _C), jnp.int32),
            pltpu.VMEM((_CH, _C), jnp.int32),
            pltpu.VMEM((_C, _H), jnp.float32),
            pltpu.VMEM((_C, _H), jnp.float32),
            pltpu.VMEM((_C, _H), jnp.float32),
            pltpu.VMEM((_C, _H), jnp.float32),
            pltpu.VMEM((_ZR, _H), jnp.float32),
            pltpu.VMEM_SHARED((_NP, _H), jnp.float32),
            pltpu.SemaphoreType.DMA,
            pltpu.SemaphoreType.DMA,
            pltpu.SemaphoreType.DMA,
            pltpu.SemaphoreType.DMA,
        ],
        compiler_params=pltpu.CompilerParams(use_tc_tiling_on_sc=False),
    )


def kernel(x, edge_attr, descriptors, Wn, bnb, gn, btn, We, be, ge, bte,
           Wc0, bc0, gc0, btc0, Wc1, bc1, gc1, btc1, Wc2, bc2, gc2, btc2,
           Wd, bd, gd, btd, Wf1, bf1, Wf2, bf2, edge_index, batch):
    src3d = edge_index[0].reshape(_NW, _CH, _C)
    dst3d = edge_index[1].reshape(_NW, _CH, _C)
    batch2d = batch.reshape(1, _N)
    desc = descriptors[:, 0, :]
    ea_t = edge_attr.T

    stats = _edge_stats(ea_t, We, be)
    e2 = _edge_emit(ea_t, stats, We, be, ge, bte)
    h_sc = _node_enc(x, Wn, bnb, gn, btn)
    h128 = h_sc.reshape(_N // 2, 2 * _H)

    for (W, b, g, bt) in ((Wc0, bc0, gc0, btc0), (Wc1, bc1, gc1, btc1),
                          (Wc2, bc2, gc2, btc2)):
        parts = _sc_msg_kernel_fn()(h_sc, e2, src3d, dst3d)
        h128 = _layer_enc(h128, parts.reshape(_NP, 2 * _H), W, b, g, bt)
        h_sc = h128.reshape(_N, _H)

    out = _final(h_sc, batch2d, desc, Wd, bd, gd, btd, Wf1, bf1, Wf2, bf2)
    return out.reshape(-1)


# speedup vs baseline: 9.0751x; 1.0158x over previous
import functools

import jax
import jax.numpy as jnp
from jax import lax
from jax.experimental import pallas as pl
from jax.experimental.pallas import tpu as pltpu
from jax.experimental.pallas import tpu_sc as plsc

_N = 10000
_E = 320000
_G = 64
_DIN = 128
_DE = 16
_H = 64
_EPS = 1e-5

_NC = 2
_NS = 16
_NW = _NC * _NS
_C = 80
_EPW = _E // _NW
_CH = _EPW // _C
_NP = 10240
_RPS = _NP // _NS
_ZR = 128


def _full(shape):
    return pl.BlockSpec(shape, lambda *args: (0,) * len(shape))


def _dott(lhs_t, rhs):
    return lax.dot_general(lhs_t, rhs, (((0,), (0,)), ((), ())),
                           preferred_element_type=jnp.float32)


def _edge_stats_body(ea_ref, We_ref, be_ref, out_ref):
    i = pl.program_id(0)
    y = jnp.maximum(
        _dott(ea_ref[...].astype(jnp.bfloat16),
              We_ref[...].astype(jnp.bfloat16)) + be_ref[...], 0.0)
    part = jnp.concatenate(
        [jnp.sum(y, axis=0).reshape(1, _H),
         jnp.sum(y * y, axis=0).reshape(1, _H)], axis=0)

    @pl.when(i == 0)
    def _():
        out_ref[...] = part

    @pl.when(i > 0)
    def _():
        out_ref[...] += part


def _edge_stats(ea_t, We, be):
    nb = 50
    cols = _E // nb
    return pl.pallas_call(
        _edge_stats_body,
        grid=(nb,),
        in_specs=[
            pl.BlockSpec((_DE, cols), lambda i: (0, i)),
            _full((_DE, _H)),
            _full((_H,)),
        ],
        out_specs=_full((2, _H)),
        out_shape=jax.ShapeDtypeStruct((2, _H), jnp.float32),
    )(ea_t, We, be)


def _edge_emit_body(ea0_ref, ea1_ref, st_ref, We_ref, be_ref, ge_ref,
                    bte_ref, out_ref):
    mean = st_ref[0, :] * (1.0 / _E)
    var = st_ref[1, :] * (1.0 / _E) - mean * mean
    scale = ge_ref[...] * lax.rsqrt(var + _EPS)
    shift = bte_ref[...] - mean * scale
    y0 = jnp.maximum(_dott(ea0_ref[...], We_ref[...]) + be_ref[...], 0.0)
    y1 = jnp.maximum(_dott(ea1_ref[...], We_ref[...]) + be_ref[...], 0.0)
    e0 = jnp.maximum(y0 * scale + shift, 0.0)
    e1 = jnp.maximum(y1 * scale + shift, 0.0)
    out_ref[...] = jnp.concatenate([e0, e1], axis=1)


def _edge_emit(ea_t, stats, We, be, ge, bte):
    nb = 25
    cols = (_E // 2) // nb
    return pl.pallas_call(
        _edge_emit_body,
        grid=(nb,),
        in_specs=[
            pl.BlockSpec((_DE, cols), lambda i: (0, i)),
            pl.BlockSpec((_DE, cols), lambda i: (0, i + nb)),
            _full((2, _H)),
            _full((_DE, _H)),
            _full((_H,)),
            _full((_H,)),
            _full((_H,)),
        ],
        out_specs=pl.BlockSpec((cols, 2 * _H), lambda i: (i, 0)),
        out_shape=jax.ShapeDtypeStruct((_E // 2, 2 * _H), jnp.float32),
    )(ea_t, ea_t, stats, We, be, ge, bte)


def _enc_rows(z, W, b, g, bt):
    y = jnp.maximum(
        jnp.dot(z, W, preferred_element_type=jnp.float32) + b, 0.0)
    m = jnp.mean(y, axis=0)
    d = y - m
    v = jnp.mean(d * d, axis=0)
    return jnp.maximum(d * lax.rsqrt(v + _EPS) * g + bt, 0.0)


def _node_enc_body(x_ref, W_ref, b_ref, g_ref, bt_ref, out_ref):
    out_ref[...] = _enc_rows(x_ref[...], W_ref[...], b_ref[...], g_ref[...],
                             bt_ref[...])


def _node_enc(x, Wn, bnb, gn, btn):
    return pl.pallas_call(
        _node_enc_body,
        in_specs=[_full((_N, _DIN)), _full((_DIN, _H)), _full((_H,)),
                  _full((_H,)), _full((_H,))],
        out_specs=_full((_N, _H)),
        out_shape=jax.ShapeDtypeStruct((_N, _H), jnp.float32),
    )(x, Wn, bnb, gn, btn)


def _layer_enc_body(h_ref, parts_ref, W_ref, b_ref, g_ref, bt_ref, out_ref):
    z = (h_ref[...] + parts_ref[:_N // 2, :]
         + parts_ref[_NP // 2:_NP // 2 + _N // 2, :])
    zero = jnp.zeros((_H, _H), jnp.float32)
    W2 = jnp.concatenate(
        [jnp.concatenate([W_ref[...], zero], axis=1),
         jnp.concatenate([zero, W_ref[...]], axis=1)], axis=0)
    b2 = jnp.concatenate([b_ref[...], b_ref[...]])
    y = jnp.maximum(
        jnp.dot(z, W2, preferred_element_type=jnp.float32) + b2, 0.0)
    s = jnp.sum(y, axis=0)
    m64 = (s[:_H] + s[_H:]) * (1.0 / _N)
    d = y - jnp.concatenate([m64, m64])
    sq = jnp.sum(d * d, axis=0)
    v64 = (sq[:_H] + sq[_H:]) * (1.0 / _N)
    r = jnp.concatenate([lax.rsqrt(v64 + _EPS)] * 2)
    g2 = jnp.concatenate([g_ref[...], g_ref[...]])
    bt2 = jnp.concatenate([bt_ref[...], bt_ref[...]])
    out_ref[...] = jnp.maximum(d * r * g2 + bt2, 0.0)


def _layer_enc(h128, parts128, W, b, g, bt):
    return pl.pallas_call(
        _layer_enc_body,
        in_specs=[_full((_N // 2, 2 * _H)), _full((_NP, 2 * _H)),
                  _full((_H, _H)), _full((_H,)), _full((_H,)), _full((_H,))],
        out_specs=_full((_N // 2, 2 * _H)),
        out_shape=jax.ShapeDtypeStruct((_N // 2, 2 * _H), jnp.float32),
    )(h128, parts128, W, b, g, bt)


def _final_body(h_ref, batch_ref, desc_ref, Wd_ref, bd_ref, gd_ref, btd_ref,
                Wf1_ref, bf1_ref, Wf2_ref, bf2_ref, out_ref):
    seg = lax.broadcasted_iota(jnp.int32, (_G, _N), 0)
    onehot = jnp.where(seg == batch_ref[0, :][None, :], 1.0, 0.0)
    pooled = jnp.dot(onehot, h_ref[...], preferred_element_type=jnp.float32)
    demb = _enc_rows(desc_ref[...], Wd_ref[...], bd_ref[...], gd_ref[...],
                     btd_ref[...])
    comb = jnp.concatenate([pooled, demb], axis=1)
    o = jnp.maximum(
        jnp.dot(comb, Wf1_ref[...], preferred_element_type=jnp.float32)
        + bf1_ref[...], 0.0)
    o2 = jnp.dot(o, Wf2_ref[...], preferred_element_type=jnp.float32) \
        + bf2_ref[...]
    out_ref[...] = 1.0 / (1.0 + jnp.exp(-o2))


def _final(h, batch2d, desc, Wd, bd, gd, btd, Wf1, bf1, Wf2, bf2):
    return pl.pallas_call(
        _final_body,
        in_specs=[_full((_N, _H)), _full((1, _N)), _full((_G, _DE)),
                  _full((_DE, _H)), _full((_H,)), _full((_H,)), _full((_H,)),
                  _full((2 * _H, _H)), _full((_H,)), _full((_H, 1)),
                  _full((1,))],
        out_specs=_full((_G, 1)),
        out_shape=jax.ShapeDtypeStruct((_G, 1), jnp.float32),
    )(h, batch2d, desc, Wd, bd, gd, btd, Wf1, bf1, Wf2, bf2)


def _sc_msg_body(h_hbm, e_hbm, src_hbm, dst_hbm, out_hbm,
                 src_v, dst_v, rows_a, e_a, msg_a, rows_b, e_b, msg_b,
                 zbuf, aggr_sh, sem_ga, sem_ea, sem_gb, sem_eb):
    c = lax.axis_index("c")
    s = lax.axis_index("s")
    wid = c * _NS + s

    pltpu.sync_copy(src_hbm.at[wid], src_v)
    pltpu.sync_copy(dst_hbm.at[wid], dst_v)

    def zrow(i, carry):
        for j in range(_H // 16):
            zbuf[i, pl.ds(j * 16, 16)] = jnp.zeros((16,), jnp.float32)
        return carry
    lax.fori_loop(0, _ZR, zrow, 0)
    for k in range(_RPS // _ZR):
        pltpu.sync_copy(zbuf, aggr_sh.at[pl.ds(s * _RPS + k * _ZR, _ZR)])
    plsc.subcore_barrier()

    ebase = s * _EPW
    ecol = c * _H

    def issue(g, rows_v, e_v, sem_g, sem_e):
        pltpu.async_copy(h_hbm.at[src_v.at[g]], rows_v, sem_g)
        pltpu.async_copy(
            e_hbm.at[pl.ds(ebase + g * _C, _C), pl.ds(ecol, _H)], e_v, sem_e)

    def process(g, rows_v, e_v, msg_v, sem_g, sem_e):
        pltpu.make_async_copy(h_hbm.at[src_v.at[g]], rows_v, sem_g).wait()
        pltpu.make_async_copy(
            e_hbm.at[pl.ds(0, _C), pl.ds(ecol, _H)], e_v, sem_e).wait()

        def _rowq(i, rcarry):
            i4 = i * 4
            for r in range(4):
                for j in range(_H // 16):
                    sl = pl.ds(j * 16, 16)
                    msg_v[i4 + r, sl] = jnp.maximum(
                        rows_v[i4 + r, sl] + e_v[i4 + r, sl], 0.0)
            return rcarry
        lax.fori_loop(0, _C // 4, _rowq, 0)
        pltpu.sync_copy(msg_v, aggr_sh.at[dst_v.at[g]], add=True)

    issue(0, rows_a, e_a, sem_ga, sem_ea)

    def pair(i, carry):
        ga = 2 * i
        issue(ga + 1, rows_b, e_b, sem_gb, sem_eb)
        process(ga, rows_a, e_a, msg_a, sem_ga, sem_ea)
        issue(ga + 2, rows_a, e_a, sem_ga, sem_ea)
        process(ga + 1, rows_b, e_b, msg_b, sem_gb, sem_eb)
        return carry
    lax.fori_loop(0, (_CH - 1) // 2, pair, 0)
    process(_CH - 1, rows_a, e_a, msg_a, sem_ga, sem_ea)

    plsc.subcore_barrier()
    pltpu.sync_copy(aggr_sh.at[pl.ds(s * _RPS, _RPS)],
                    out_hbm.at[c, pl.ds(s * _RPS, _RPS)])


@functools.lru_cache(maxsize=1)
def _sc_msg_kernel_fn():
    mesh = plsc.VectorSubcoreMesh(core_axis_name="c", subcore_axis_name="s",
                                  num_cores=_NC, num_subcores=_NS)
    return pl.kernel(
        _sc_msg_body,
        out_type=jax.ShapeDtypeStruct((2, _NP, _H), jnp.float32),
        mesh=mesh,
        scratch_types=[
            pltpu.VMEM((_CH, _C), jnp.int32),
            pltpu.VMEM((_CH, _C), jnp.int32),
            pltpu.VMEM((_C, _H), jnp.float32),
            pltpu.VMEM((_C, _H), jnp.float32),
            pltpu.VMEM((_C, _H), jnp.float32),
            pltpu.VMEM((_C, _H), jnp.float32),
            pltpu.VMEM((_C, _H), jnp.float32),
            pltpu.VMEM((_C, _H), jnp.float32),
            pltpu.VMEM((_ZR, _H), jnp.float32),
            pltpu.VMEM_SHARED((_NP, _H), jnp.float32),
            pltpu.SemaphoreType.DMA,
            pltpu.SemaphoreType.DMA,
            pltpu.SemaphoreType.DMA,
            pltpu.SemaphoreType.DMA,
        ],
        compiler_params=pltpu.CompilerParams(use_tc_tiling_on_sc=False),
    )


def kernel(x, edge_attr, descriptors, Wn, bnb, gn, btn, We, be, ge, bte,
           Wc0, bc0, gc0, btc0, Wc1, bc1, gc1, btc1, Wc2, bc2, gc2, btc2,
           Wd, bd, gd, btd, Wf1, bf1, Wf2, bf2, edge_index, batch):
    src3d = edge_index[0].reshape(_NW, _CH, _C)
    dst3d = edge_index[1].reshape(_NW, _CH, _C)
    batch2d = batch.reshape(1, _N)
    desc = descriptors[:, 0, :]
    ea_t = edge_attr.T

    stats = _edge_stats(ea_t, We, be)
    e2 = _edge_emit(ea_t, stats, We, be, ge, bte)
    h_sc = _node_enc(x, Wn, bnb, gn, btn)
    h128 = h_sc.reshape(_N // 2, 2 * _H)

    for (W, b, g, bt) in ((Wc0, bc0, gc0, btc0), (Wc1, bc1, gc1, btc1),
                          (Wc2, bc2, gc2, btc2)):
        parts = _sc_msg_kernel_fn()(h_sc, e2, src3d, dst3d)
        h128 = _layer_enc(h128, parts.reshape(_NP, 2 * _H), W, b, g, bt)
        h_sc = h128.reshape(_N, _H)

    out = _final(h_sc, batch2d, desc, Wd, bd, gd, btd, Wf1, bf1, Wf2, bf2)
    return out.reshape(-1)
